# baseline (speedup 1.0000x reference)
import functools

import jax
import jax.numpy as jnp
from jax import lax
from jax.experimental import pallas as pl
from jax.experimental.pallas import tpu as pltpu
from jax.experimental.pallas import tpu_sc as plsc

_B = 4096
_S = 200
_D = 32
_V = 1000000
_NW = 32
_TILE = 4096


def _lk_body(
    xv, tab, pos_hbm, out,
    xidx, pos_v,
    r0, r1, r2, r3, r4, r5, r6, r7,
    t0, t1, t2, t3, t4, t5, t6, t7,
    g0, g1, g2, g3, g4, g5, g6, g7,
    w0, w1, w2, w3, w4, w5, w6, w7,
):
    cid = lax.axis_index("c")
    sid = lax.axis_index("s")
    wid = sid * 2 + cid

    rows = (r0, r1, r2, r3, r4, r5, r6, r7)
    tiles = (t0, t1, t2, t3, t4, t5, t6, t7)
    gsems = (g0, g1, g2, g3, g4, g5, g6, g7)
    wsems = (w0, w1, w2, w3, w4, w5, w6, w7)

    pltpu.sync_copy(pos_hbm, pos_v)
    pltpu.sync_copy(xv.at[:, wid], xidx)

    iota = lax.iota(jnp.int32, 16)
    pat = iota * 128

    def fire_gather(s, j):
        pltpu.async_copy(
            tab.at[xidx.at[s // 8, lax.rem(s, 8)]], rows[j], gsems[j]
        )

    def drain(dst, sem, src):
        pltpu.make_async_copy(src, dst, sem).wait()

    for j in range(8):
        fire_gather(jnp.int32(j), j)

    @pl.loop(0, _S // 8)
    def _k(k):
        for j in range(8):
            s = k * 8 + j
            drain(rows[j], gsems[j], tab.at[pl.ds(0, 128)])

            @pl.when(k > 0)
            def _():
                drain(out.at[pl.ds(0, _TILE)], wsems[j], tiles[j])

            pv0 = pos_v[s, pl.ds(0, 16)]
            pv1 = pos_v[s, pl.ds(16, 16)]

            @plsc.parallel_loop(0, 128, unroll=8)
            def _bc(bc):
                v0 = rows[j][bc, pl.ds(0, 16)] + pv0
                v1 = rows[j][bc, pl.ds(16, 16)] + pv1
                plsc.store_scatter(tiles[j], [pat + bc], v0)
                plsc.store_scatter(tiles[j], [pat + (bc + 2048)], v1)

            base = (s * 4 * _NW + wid) * 1024
            for dq in range(4):
                pltpu.async_copy(
                    tiles[j].at[pl.ds(dq * 1024, 1024)],
                    out.at[pl.ds(base + dq * _NW * 1024, 1024)],
                    wsems[j],
                )

            @pl.when(k < _S // 8 - 1)
            def _():
                fire_gather(s + 8, j)

    for j in range(8):
        drain(out.at[pl.ds(0, _TILE)], wsems[j], tiles[j])


@jax.jit
def _emb(x, token_table, pos_table):
    mesh = plsc.VectorSubcoreMesh(
        core_axis_name="c", subcore_axis_name="s", num_cores=2, num_subcores=16
    )
    cp = pltpu.CompilerParams(
        use_tc_tiling_on_sc=False, needs_layout_passes=False
    )

    f_lk = pl.kernel(
        _lk_body,
        out_type=jax.ShapeDtypeStruct((_S * 4 * _NW * 1024,), jnp.float32),
        mesh=mesh,
        scratch_types=(
            [
                pltpu.VMEM((25, 8, 128), jnp.int32),
                pltpu.VMEM((_S, _D), jnp.float32),
            ]
            + [pltpu.VMEM((128, _D), jnp.float32)] * 8
            + [pltpu.VMEM((_TILE,), jnp.float32)] * 8
            + [pltpu.SemaphoreType.DMA] * 16
        ),
        compiler_params=cp,
    )

    xv = x.T.reshape(25, 8, 32, 128).transpose(0, 2, 1, 3)
    V = f_lk(xv, token_table, pos_table)
    return (
        V.reshape(_S, 4, 32, 8, 128)
        .transpose(2, 4, 0, 1, 3)
        .reshape(_B, _S, _D)
    )


def kernel(x, token_table, pos_table):
    return _emb(x, token_table, pos_table)


# speedup vs baseline: 1.0418x; 1.0418x over previous
import functools

import jax
import jax.numpy as jnp
from jax import lax
from jax.experimental import pallas as pl
from jax.experimental.pallas import tpu as pltpu
from jax.experimental.pallas import tpu_sc as plsc

_B = 4096
_S = 200
_D = 32
_V = 1000000
_NW = 32


def _lk_body(
    xv, tab, pos_hbm, out,
    xidx, pos_v,
    r0, r1, r2, r3, t0, t1, t2, t3,
    g0, g1, g2, g3, w0, w1, w2, w3,
):
    cid = lax.axis_index("c")
    sid = lax.axis_index("s")
    wid = sid * 2 + cid

    rows = (r0, r1, r2, r3)
    tiles = (t0, t1, t2, t3)
    gsems = (g0, g1, g2, g3)
    wsems = (w0, w1, w2, w3)

    pltpu.sync_copy(pos_hbm, pos_v)
    pltpu.sync_copy(xv.at[:, wid], xidx)

    iota = lax.iota(jnp.int32, 16)
    bcs = [iota + 16 * b for b in range(8)]

    def fire_gather(s, j):
        pltpu.async_copy(
            tab.at[xidx.at[s // 8, lax.rem(s, 8)]], rows[j], gsems[j]
        )

    def drain(dst, sem, src):
        pltpu.make_async_copy(src, dst, sem).wait()

    for j in range(4):
        fire_gather(jnp.int32(j), j)

    @pl.loop(0, _S // 4)
    def _k(k):
        for j in range(4):
            s = k * 4 + j
            drain(rows[j], gsems[j], tab.at[pl.ds(0, 128)])

            @pl.when(k > 0)
            def _():
                drain(out.at[0, :, 0], wsems[j], tiles[j])

            ssp = jnp.full((16,), s, jnp.int32)

            @plsc.parallel_loop(0, _D, unroll=16)
            def _d(d):
                dsp = jnp.full((16,), d, jnp.int32)
                ps = plsc.load_gather(pos_v, [ssp, dsp])
                dq = d // 8
                dr = lax.rem(d, 8)
                for b in range(8):
                    v = plsc.load_gather(rows[j], [bcs[b], dsp]) + ps
                    tiles[j][dq, dr, pl.ds(16 * b, 16)] = v

            pltpu.async_copy(tiles[j], out.at[s, :, wid], wsems[j])

            @pl.when(k < _S // 4 - 1)
            def _():
                fire_gather(s + 4, j)

    for j in range(4):
        drain(out.at[0, :, 0], wsems[j], tiles[j])


@jax.jit
def _emb(x, token_table, pos_table):
    mesh = plsc.VectorSubcoreMesh(
        core_axis_name="c", subcore_axis_name="s", num_cores=2, num_subcores=16
    )
    cp = pltpu.CompilerParams(
        use_tc_tiling_on_sc=False, needs_layout_passes=False
    )

    f_lk = pl.kernel(
        _lk_body,
        out_type=jax.ShapeDtypeStruct((_S, 4, 32, 8, 128), jnp.float32),
        mesh=mesh,
        scratch_types=(
            [
                pltpu.VMEM((25, 8, 128), jnp.int32),
                pltpu.VMEM((_S, _D), jnp.float32),
            ]
            + [pltpu.VMEM((128, _D), jnp.float32)] * 4
            + [pltpu.VMEM((4, 8, 128), jnp.float32)] * 4
            + [pltpu.SemaphoreType.DMA] * 8
        ),
        compiler_params=cp,
    )

    xv = x.T.reshape(25, 8, 32, 128).transpose(0, 2, 1, 3)
    V = f_lk(xv, token_table, pos_table)
    return V.transpose(2, 4, 0, 1, 3).reshape(_B, _S, _D)


def kernel(x, token_table, pos_table):
    return _emb(x, token_table, pos_table)
